# baseline (speedup 1.0000x reference)
import jax
import jax.numpy as jnp
from jax import lax
from jax.experimental import pallas as pl

_B, _S, _D, _H, _K = 2, 2048, 768, 12, 16
_HD = _D // _H
_SCALE = _HD ** (-0.5)

_RB = 256
_QB = 256
_BIG = 3.0e38
_NEG = -1.0e30


def _proj_body(x_ref, wq_ref, wk_ref, wv_ref, q_ref, k_ref, v_ref, m_ref):
    x = x_ref[0].astype(jnp.bfloat16)
    wq = wq_ref[...].astype(jnp.bfloat16)
    wk = wk_ref[...].astype(jnp.bfloat16)
    wv = wv_ref[...].astype(jnp.bfloat16)
    dn = (((1,), (1,)), ((), ()))
    q = lax.dot_general(x, wq, dn, preferred_element_type=jnp.float32)
    k = lax.dot_general(x, wk, dn, preferred_element_type=jnp.float32)
    v = lax.dot_general(x, wv, dn, preferred_element_type=jnp.float32)
    q_ref[0] = (q * _SCALE).astype(jnp.bfloat16)
    k_ref[0] = k.astype(jnp.bfloat16)
    v_ref[0] = v.astype(jnp.bfloat16)
    acc = jnp.zeros((x.shape[0], _HD), jnp.float32)
    for h in range(_H):
        acc = acc + k[:, h * _HD:(h + 1) * _HD]
    m_ref[0] = acc * (1.0 / _H)


def _attn_body(q_ref, k_ref, v_ref, ck_ref, cqt_ref, out_ref):
    jq = pl.program_id(1)
    ck = ck_ref[0]
    cqt = cqt_ref[0]
    kn = (ck[:, 0:1] * ck[:, 0:1] + ck[:, 1:2] * ck[:, 1:2]
          + ck[:, 2:3] * ck[:, 2:3])
    qn = (cqt[0:1, :] * cqt[0:1, :] + cqt[1:2, :] * cqt[1:2, :]
          + cqt[2:3, :] * cqt[2:3, :])
    cross = lax.dot_general(ck.astype(jnp.bfloat16), cqt.astype(jnp.bfloat16),
                            (((1,), (0,)), ((), ())),
                            preferred_element_type=jnp.float32)
    d = (qn + kn) - 2.0 * cross
    rid = lax.broadcasted_iota(jnp.int32, (_S, _QB), 0)
    cid = jq * _QB + lax.broadcasted_iota(jnp.int32, (_S, _QB), 1)
    d = jnp.where(rid == cid, _BIG, d)

    def body(i, carry):
        dcur, _ = carry
        m = jnp.min(dcur, axis=0, keepdims=True)
        dcur = jnp.where(dcur <= m, _BIG, dcur)
        return (dcur, m)

    _, thr = lax.fori_loop(0, _K, body, (d, jnp.zeros((1, _QB), jnp.float32)))
    bias = jnp.where(d <= thr, 0.0, _NEG)

    dn_t = (((1,), (1,)), ((), ()))
    dn_c0 = (((0,), (0,)), ((), ()))
    for h in range(_H):
        sl = slice(h * _HD, (h + 1) * _HD)
        qh = q_ref[0, :, sl]
        kh = k_ref[0, :, sl]
        vh = v_ref[0, :, sl]
        st = lax.dot_general(kh, qh, dn_t, preferred_element_type=jnp.float32)
        p = jnp.exp(st + bias)
        denom = jnp.sum(p, axis=0, keepdims=True)
        pn = (p * (1.0 / denom)).astype(jnp.bfloat16)
        oh = lax.dot_general(pn, vh, dn_c0, preferred_element_type=jnp.float32)
        out_ref[0, :, sl] = oh


@jax.jit
def kernel(x, coords, Wq, Wk, Wv):
    q, k, v, metric = pl.pallas_call(
        _proj_body,
        grid=(_B, _S // _RB),
        in_specs=[
            pl.BlockSpec((1, _RB, _D), lambda b, j: (b, j, 0)),
            pl.BlockSpec((_D, _D), lambda b, j: (0, 0)),
            pl.BlockSpec((_D, _D), lambda b, j: (0, 0)),
            pl.BlockSpec((_D, _D), lambda b, j: (0, 0)),
        ],
        out_specs=[
            pl.BlockSpec((1, _RB, _D), lambda b, j: (b, j, 0)),
            pl.BlockSpec((1, _RB, _D), lambda b, j: (b, j, 0)),
            pl.BlockSpec((1, _RB, _D), lambda b, j: (b, j, 0)),
            pl.BlockSpec((1, _RB, _HD), lambda b, j: (b, j, 0)),
        ],
        out_shape=[
            jax.ShapeDtypeStruct((_B, _S, _D), jnp.bfloat16),
            jax.ShapeDtypeStruct((_B, _S, _D), jnp.bfloat16),
            jax.ShapeDtypeStruct((_B, _S, _D), jnp.bfloat16),
            jax.ShapeDtypeStruct((_B, _S, _HD), jnp.float32),
        ],
    )(x, Wq, Wk, Wv)

    coords_pad = jnp.pad(coords, ((0, 0), (0, 0), (0, 5)))
    coords_t = jnp.pad(jnp.swapaxes(coords, 1, 2),
                       ((0, 0), (0, 5), (0, 0)))

    out = pl.pallas_call(
        _attn_body,
        grid=(_B, _S // _QB),
        in_specs=[
            pl.BlockSpec((1, _QB, _D), lambda b, j: (b, j, 0)),
            pl.BlockSpec((1, _S, _D), lambda b, j: (b, 0, 0)),
            pl.BlockSpec((1, _S, _D), lambda b, j: (b, 0, 0)),
            pl.BlockSpec((1, _S, 8), lambda b, j: (b, 0, 0)),
            pl.BlockSpec((1, 8, _QB), lambda b, j: (b, 0, j)),
        ],
        out_specs=pl.BlockSpec((1, _QB, _D), lambda b, j: (b, j, 0)),
        out_shape=jax.ShapeDtypeStruct((_B, _S, _D), jnp.float32),
    )(q, k, v, coords_pad, coords_t)

    return (out, metric)


# speedup vs baseline: 1.0377x; 1.0377x over previous
import jax
import jax.numpy as jnp
from jax import lax
from jax.experimental import pallas as pl

_B, _S, _D, _H, _K = 2, 2048, 768, 12, 16
_HD = _D // _H
_SCALE = _HD ** (-0.5)

_RB = 256
_QB = 512
_BIG = 3.0e38
_NEG = -1.0e30


def _proj_body(x_ref, wq_ref, wk_ref, wv_ref, q_ref, k_ref, v_ref, m_ref):
    x = x_ref[0].astype(jnp.bfloat16)
    wq = wq_ref[...].astype(jnp.bfloat16)
    wk = wk_ref[...].astype(jnp.bfloat16)
    wv = wv_ref[...].astype(jnp.bfloat16)
    dn = (((1,), (1,)), ((), ()))
    q = lax.dot_general(x, wq, dn, preferred_element_type=jnp.float32)
    k = lax.dot_general(x, wk, dn, preferred_element_type=jnp.float32)
    v = lax.dot_general(x, wv, dn, preferred_element_type=jnp.float32)
    q_ref[0] = (q * _SCALE).astype(jnp.bfloat16)
    k_ref[0] = k.astype(jnp.bfloat16)
    v_ref[0] = v.astype(jnp.bfloat16)
    acc = jnp.zeros((x.shape[0], _HD), jnp.float32)
    for h in range(_H):
        acc = acc + k[:, h * _HD:(h + 1) * _HD]
    m_ref[0] = acc * (1.0 / _H)


def _attn_body(q_ref, k_ref, v_ref, ck_ref, cqt_ref, out_ref):
    jq = pl.program_id(1)
    ck = ck_ref[0]
    cqt = cqt_ref[0]
    kn = (ck[:, 0:1] * ck[:, 0:1] + ck[:, 1:2] * ck[:, 1:2]
          + ck[:, 2:3] * ck[:, 2:3])
    qn = (cqt[0:1, :] * cqt[0:1, :] + cqt[1:2, :] * cqt[1:2, :]
          + cqt[2:3, :] * cqt[2:3, :])
    cross = lax.dot_general(ck.astype(jnp.bfloat16), cqt.astype(jnp.bfloat16),
                            (((1,), (0,)), ((), ())),
                            preferred_element_type=jnp.float32)
    rid = lax.broadcasted_iota(jnp.int32, (_S, _QB), 0)
    cid = jq * _QB + lax.broadcasted_iota(jnp.int32, (_S, _QB), 1)
    d = jnp.where(rid == cid, _BIG, (qn + kn) - 2.0 * cross)

    def body(i, carry):
        dcur, m = carry
        dcur = jnp.where(dcur <= m, _BIG, dcur)
        return (dcur, jnp.min(dcur, axis=0, keepdims=True))

    _, thr = lax.fori_loop(0, _K, body,
                           (d, jnp.full((1, _QB), _NEG, jnp.float32)))
    m = d <= thr

    dn_t = (((1,), (1,)), ((), ()))
    dn_c0 = (((0,), (0,)), ((), ()))
    ones_col = jnp.ones((_S, 1), jnp.bfloat16)
    for h in range(_H):
        sl = slice(h * _HD, (h + 1) * _HD)
        qh = q_ref[0, :, sl]
        kh = k_ref[0, :, sl]
        vh = v_ref[0, :, sl]
        st = lax.dot_general(kh, qh, dn_t, preferred_element_type=jnp.float32)
        pb = jnp.exp(jnp.where(m, st, _NEG)).astype(jnp.bfloat16)
        denom = lax.dot_general(pb, ones_col, dn_c0,
                                preferred_element_type=jnp.float32)
        oh = lax.dot_general(pb, vh, dn_c0, preferred_element_type=jnp.float32)
        out_ref[0, :, sl] = oh * (1.0 / denom)


@jax.jit
def kernel(x, coords, Wq, Wk, Wv):
    q, k, v, metric = pl.pallas_call(
        _proj_body,
        grid=(_B, _S // _RB),
        in_specs=[
            pl.BlockSpec((1, _RB, _D), lambda b, j: (b, j, 0)),
            pl.BlockSpec((_D, _D), lambda b, j: (0, 0)),
            pl.BlockSpec((_D, _D), lambda b, j: (0, 0)),
            pl.BlockSpec((_D, _D), lambda b, j: (0, 0)),
        ],
        out_specs=[
            pl.BlockSpec((1, _RB, _D), lambda b, j: (b, j, 0)),
            pl.BlockSpec((1, _RB, _D), lambda b, j: (b, j, 0)),
            pl.BlockSpec((1, _RB, _D), lambda b, j: (b, j, 0)),
            pl.BlockSpec((1, _RB, _HD), lambda b, j: (b, j, 0)),
        ],
        out_shape=[
            jax.ShapeDtypeStruct((_B, _S, _D), jnp.bfloat16),
            jax.ShapeDtypeStruct((_B, _S, _D), jnp.bfloat16),
            jax.ShapeDtypeStruct((_B, _S, _D), jnp.bfloat16),
            jax.ShapeDtypeStruct((_B, _S, _HD), jnp.float32),
        ],
    )(x, Wq, Wk, Wv)

    coords_pad = jnp.pad(coords, ((0, 0), (0, 0), (0, 5)))
    coords_t = jnp.pad(jnp.swapaxes(coords, 1, 2),
                       ((0, 0), (0, 5), (0, 0)))

    out = pl.pallas_call(
        _attn_body,
        grid=(_B, _S // _QB),
        in_specs=[
            pl.BlockSpec((1, _QB, _D), lambda b, j: (b, j, 0)),
            pl.BlockSpec((1, _S, _D), lambda b, j: (b, 0, 0)),
            pl.BlockSpec((1, _S, _D), lambda b, j: (b, 0, 0)),
            pl.BlockSpec((1, _S, 8), lambda b, j: (b, 0, 0)),
            pl.BlockSpec((1, 8, _QB), lambda b, j: (b, 0, j)),
        ],
        out_specs=pl.BlockSpec((1, _QB, _D), lambda b, j: (b, j, 0)),
        out_shape=jax.ShapeDtypeStruct((_B, _S, _D), jnp.float32),
    )(q, k, v, coords_pad, coords_t)

    return (out, metric)


# speedup vs baseline: 1.2214x; 1.1770x over previous
import jax
import jax.numpy as jnp
from jax import lax
from jax.experimental import pallas as pl

_B, _S, _D, _H, _K = 2, 2048, 768, 12, 16
_HD = _D // _H
_SCALE = _HD ** (-0.5)

_RB = 256
_QB = 512
_BIG = 3.0e38
_NEG = -1.0e30


def _proj_body(x_ref, wq_ref, wk_ref, wv_ref, q_ref, k_ref, v_ref, m_ref):
    x = x_ref[0].astype(jnp.bfloat16)
    wq = wq_ref[...].astype(jnp.bfloat16)
    wk = wk_ref[...].astype(jnp.bfloat16)
    wv = wv_ref[...].astype(jnp.bfloat16)
    dn = (((1,), (1,)), ((), ()))
    q = lax.dot_general(x, wq, dn, preferred_element_type=jnp.float32)
    k = lax.dot_general(x, wk, dn, preferred_element_type=jnp.float32)
    v = lax.dot_general(x, wv, dn, preferred_element_type=jnp.float32)
    q_ref[0] = (q * _SCALE).astype(jnp.bfloat16)
    k_ref[0] = k.astype(jnp.bfloat16)
    v_ref[0] = v.astype(jnp.bfloat16)
    acc = jnp.zeros((x.shape[0], _HD), jnp.float32)
    for h in range(_H):
        acc = acc + k[:, h * _HD:(h + 1) * _HD]
    m_ref[0] = acc * (1.0 / _H)


def _attn_body(q_ref, k_ref, v_ref, ck_ref, cqt_ref, out_ref):
    jq = pl.program_id(1)
    ck = ck_ref[0]
    cqt = cqt_ref[0]
    kn = (ck[:, 0:1] * ck[:, 0:1] + ck[:, 1:2] * ck[:, 1:2]
          + ck[:, 2:3] * ck[:, 2:3])
    qn = (cqt[0:1, :] * cqt[0:1, :] + cqt[1:2, :] * cqt[1:2, :]
          + cqt[2:3, :] * cqt[2:3, :])
    cross = lax.dot_general(ck.astype(jnp.bfloat16), cqt.astype(jnp.bfloat16),
                            (((1,), (0,)), ((), ())),
                            preferred_element_type=jnp.float32)
    rid = lax.broadcasted_iota(jnp.int32, (_S, _QB), 0)
    cid = jq * _QB + lax.broadcasted_iota(jnp.int32, (_S, _QB), 1)
    d = jnp.where(rid == cid, _BIG, (qn + kn) - 2.0 * cross)

    def body(i, carry):
        dcur, m = carry
        dcur = jnp.where(dcur <= m, _BIG, dcur)
        return (dcur, jnp.min(dcur, axis=0, keepdims=True))

    _, thr = lax.fori_loop(0, _K, body,
                           (d, jnp.full((1, _QB), _NEG, jnp.float32)))
    m = d <= thr

    dn_t = (((1,), (1,)), ((), ()))
    dn_c0 = (((0,), (0,)), ((), ()))
    for h in range(_H):
        sl = slice(h * _HD, (h + 1) * _HD)
        qh = q_ref[0, :, sl]
        kh = k_ref[0, :, sl]
        vh = v_ref[0, :, sl]
        st = lax.dot_general(kh, qh, dn_t, preferred_element_type=jnp.float32)
        pb = jnp.exp(jnp.where(m, st, _NEG)).astype(jnp.bfloat16)
        denom = jnp.sum(pb.astype(jnp.float32), axis=0, keepdims=True)
        oh = lax.dot_general(pb, vh, dn_c0, preferred_element_type=jnp.float32)
        out_ref[0, :, sl] = oh * (1.0 / denom).T


@jax.jit
def kernel(x, coords, Wq, Wk, Wv):
    q, k, v, metric = pl.pallas_call(
        _proj_body,
        grid=(_B, _S // _RB),
        in_specs=[
            pl.BlockSpec((1, _RB, _D), lambda b, j: (b, j, 0)),
            pl.BlockSpec((_D, _D), lambda b, j: (0, 0)),
            pl.BlockSpec((_D, _D), lambda b, j: (0, 0)),
            pl.BlockSpec((_D, _D), lambda b, j: (0, 0)),
        ],
        out_specs=[
            pl.BlockSpec((1, _RB, _D), lambda b, j: (b, j, 0)),
            pl.BlockSpec((1, _RB, _D), lambda b, j: (b, j, 0)),
            pl.BlockSpec((1, _RB, _D), lambda b, j: (b, j, 0)),
            pl.BlockSpec((1, _RB, _HD), lambda b, j: (b, j, 0)),
        ],
        out_shape=[
            jax.ShapeDtypeStruct((_B, _S, _D), jnp.bfloat16),
            jax.ShapeDtypeStruct((_B, _S, _D), jnp.bfloat16),
            jax.ShapeDtypeStruct((_B, _S, _D), jnp.bfloat16),
            jax.ShapeDtypeStruct((_B, _S, _HD), jnp.float32),
        ],
    )(x, Wq, Wk, Wv)

    coords_pad = jnp.pad(coords, ((0, 0), (0, 0), (0, 5)))
    coords_t = jnp.pad(jnp.swapaxes(coords, 1, 2),
                       ((0, 0), (0, 5), (0, 0)))

    out = pl.pallas_call(
        _attn_body,
        grid=(_B, _S // _QB),
        in_specs=[
            pl.BlockSpec((1, _QB, _D), lambda b, j: (b, j, 0)),
            pl.BlockSpec((1, _S, _D), lambda b, j: (b, 0, 0)),
            pl.BlockSpec((1, _S, _D), lambda b, j: (b, 0, 0)),
            pl.BlockSpec((1, _S, 8), lambda b, j: (b, 0, 0)),
            pl.BlockSpec((1, 8, _QB), lambda b, j: (b, 0, j)),
        ],
        out_specs=pl.BlockSpec((1, _QB, _D), lambda b, j: (b, j, 0)),
        out_shape=jax.ShapeDtypeStruct((_B, _S, _D), jnp.float32),
    )(q, k, v, coords_pad, coords_t)

    return (out, metric)


# speedup vs baseline: 1.2333x; 1.0098x over previous
import jax
import jax.numpy as jnp
from jax import lax
from jax.experimental import pallas as pl

_B, _S, _D, _H, _K = 2, 2048, 768, 12, 16
_HD = _D // _H
_SCALE = _HD ** (-0.5)

_LOG2E = 1.4426950408889634

_RB = 256
_QB = 512
_BIG = 3.0e38
_NEG = -1.0e30


def _proj_body(x_ref, wq_ref, wk_ref, wv_ref, q_ref, k_ref, v_ref, m_ref):
    x = x_ref[0].astype(jnp.bfloat16)
    wq = wq_ref[...].astype(jnp.bfloat16)
    wk = wk_ref[...].astype(jnp.bfloat16)
    wv = wv_ref[...].astype(jnp.bfloat16)
    dn = (((1,), (1,)), ((), ()))
    q = lax.dot_general(x, wq, dn, preferred_element_type=jnp.float32)
    k = lax.dot_general(x, wk, dn, preferred_element_type=jnp.float32)
    v = lax.dot_general(x, wv, dn, preferred_element_type=jnp.float32)
    q_ref[0] = (q * (_SCALE * _LOG2E)).astype(jnp.bfloat16)
    k_ref[0] = k.astype(jnp.bfloat16)
    v_ref[0] = v.astype(jnp.bfloat16)
    acc = jnp.zeros((x.shape[0], _HD), jnp.float32)
    for h in range(_H):
        acc = acc + k[:, h * _HD:(h + 1) * _HD]
    m_ref[0] = acc * (1.0 / _H)


def _attn_body(q_ref, k_ref, v_ref, ck_ref, cqt_ref, out_ref):
    jq = pl.program_id(1)
    ck = ck_ref[0]
    cqt = cqt_ref[0]
    kn = (ck[:, 0:1] * ck[:, 0:1] + ck[:, 1:2] * ck[:, 1:2]
          + ck[:, 2:3] * ck[:, 2:3])
    qn = (cqt[0:1, :] * cqt[0:1, :] + cqt[1:2, :] * cqt[1:2, :]
          + cqt[2:3, :] * cqt[2:3, :])
    cross = lax.dot_general(ck.astype(jnp.bfloat16), cqt.astype(jnp.bfloat16),
                            (((1,), (0,)), ((), ())),
                            preferred_element_type=jnp.float32)
    rid = lax.broadcasted_iota(jnp.int32, (_S, _QB), 0)
    cid = jq * _QB + lax.broadcasted_iota(jnp.int32, (_S, _QB), 1)
    d = jnp.where(rid == cid, _BIG, (qn + kn) - 2.0 * cross)

    def body(i, carry):
        dcur, m = carry
        dcur = jnp.where(dcur <= m, _BIG, dcur)
        return (dcur, jnp.min(dcur, axis=0, keepdims=True))

    _, thr = lax.fori_loop(0, _K, body,
                           (d, jnp.full((1, _QB), _NEG, jnp.float32)))
    m = d <= thr

    dn_t = (((1,), (1,)), ((), ()))
    dn_c0 = (((0,), (0,)), ((), ()))
    for h in range(_H):
        sl = slice(h * _HD, (h + 1) * _HD)
        qh = q_ref[0, :, sl]
        kh = k_ref[0, :, sl]
        vh = v_ref[0, :, sl]
        st = lax.dot_general(kh, qh, dn_t, preferred_element_type=jnp.float32)
        p32 = jnp.exp2(jnp.where(m, st, _NEG))
        denom = jnp.sum(p32, axis=0, keepdims=True)
        pb = p32.astype(jnp.bfloat16)
        oh = lax.dot_general(pb, vh, dn_c0, preferred_element_type=jnp.float32)
        out_ref[0, :, sl] = oh * (1.0 / denom).T


@jax.jit
def kernel(x, coords, Wq, Wk, Wv):
    q, k, v, metric = pl.pallas_call(
        _proj_body,
        grid=(_B, _S // _RB),
        in_specs=[
            pl.BlockSpec((1, _RB, _D), lambda b, j: (b, j, 0)),
            pl.BlockSpec((_D, _D), lambda b, j: (0, 0)),
            pl.BlockSpec((_D, _D), lambda b, j: (0, 0)),
            pl.BlockSpec((_D, _D), lambda b, j: (0, 0)),
        ],
        out_specs=[
            pl.BlockSpec((1, _RB, _D), lambda b, j: (b, j, 0)),
            pl.BlockSpec((1, _RB, _D), lambda b, j: (b, j, 0)),
            pl.BlockSpec((1, _RB, _D), lambda b, j: (b, j, 0)),
            pl.BlockSpec((1, _RB, _HD), lambda b, j: (b, j, 0)),
        ],
        out_shape=[
            jax.ShapeDtypeStruct((_B, _S, _D), jnp.bfloat16),
            jax.ShapeDtypeStruct((_B, _S, _D), jnp.bfloat16),
            jax.ShapeDtypeStruct((_B, _S, _D), jnp.bfloat16),
            jax.ShapeDtypeStruct((_B, _S, _HD), jnp.float32),
        ],
    )(x, Wq, Wk, Wv)

    coords_pad = jnp.pad(coords, ((0, 0), (0, 0), (0, 5)))
    coords_t = jnp.pad(jnp.swapaxes(coords, 1, 2),
                       ((0, 0), (0, 5), (0, 0)))

    out = pl.pallas_call(
        _attn_body,
        grid=(_B, _S // _QB),
        in_specs=[
            pl.BlockSpec((1, _QB, _D), lambda b, j: (b, j, 0)),
            pl.BlockSpec((1, _S, _D), lambda b, j: (b, 0, 0)),
            pl.BlockSpec((1, _S, _D), lambda b, j: (b, 0, 0)),
            pl.BlockSpec((1, _S, 8), lambda b, j: (b, 0, 0)),
            pl.BlockSpec((1, 8, _QB), lambda b, j: (b, 0, j)),
        ],
        out_specs=pl.BlockSpec((1, _QB, _D), lambda b, j: (b, j, 0)),
        out_shape=jax.ShapeDtypeStruct((_B, _S, _D), jnp.float32),
    )(q, k, v, coords_pad, coords_t)

    return (out, metric)
